# baseline (speedup 1.0000x reference)
import functools

import jax
import jax.numpy as jnp
from jax import lax
from jax.experimental import pallas as pl
from jax.experimental.pallas import tpu as pltpu
from jax.experimental.pallas import tpu_sc as plsc

NUM_CORES = 2
NUM_SUBCORES = 16
NUM_WORKERS = NUM_CORES * NUM_SUBCORES
TB = 128
K = 8
G = 4
LANES = 16


@functools.partial(jax.jit, static_argnums=(2, 3))
def _embedding_lookup(idx2d, table, seq, tblocks):
    dim = table.shape[1]
    dg = dim // 8
    n = idx2d.shape[0] // NUM_WORKERS
    assert n % K == 0
    mesh = plsc.VectorSubcoreMesh(core_axis_name="c", subcore_axis_name="s")

    @functools.partial(
        pl.kernel,
        mesh=mesh,
        out_type=jax.ShapeDtypeStruct((seq, dg, tblocks, 8, TB), jnp.float32),
        scratch_types=[
            pltpu.VMEM((n, TB), jnp.int32),
            pltpu.VMEM((K, TB, dim), jnp.float32),
            pltpu.VMEM((2, dim, TB), jnp.float32),
            pltpu.SemaphoreType.DMA((K,)),
            pltpu.SemaphoreType.DMA((2,)),
        ],
        compiler_params=pltpu.CompilerParams(
            use_tc_tiling_on_sc=False, needs_layout_passes=False
        ),
    )
    def emb_kernel(idx_hbm, table_hbm, out_hbm, idx_v, rows_v, trans_v,
                   gsem, wsem):
        wid = lax.axis_index("s") * NUM_CORES + lax.axis_index("c")
        blk_base = wid * n
        pltpu.sync_copy(idx_hbm.at[pl.ds(blk_base, n)], idx_v)

        def fire_gather(j, b):
            pltpu.async_copy(table_hbm.at[idx_v.at[j]], rows_v.at[b], gsem.at[b])

        def wait_gather(j, b):
            pltpu.make_async_copy(
                table_hbm.at[idx_v.at[j]], rows_v.at[b], gsem.at[b]
            ).wait()

        def write_parts(j, p):
            blk = blk_base + j
            s = blk // tblocks
            tb = blk % tblocks
            return [
                (trans_v.at[p, pl.ds(8 * g, 8)], out_hbm.at[s, g, tb])
                for g in range(dg)
            ]

        def fire_writes(j, p):
            for src, dst in write_parts(j, p):
                pltpu.async_copy(src, dst, wsem.at[p])

        def wait_writes(j, p):
            for src, dst in write_parts(j, p):
                pltpu.make_async_copy(src, dst, wsem.at[p]).wait()

        lane_iota = lax.iota(jnp.int32, LANES)

        def transpose_block(b, p):
            def drow(d, _):
                for tc in range(TB // LANES):
                    tvec = lane_iota + (tc * LANES)
                    dvec = jnp.full((LANES,), 0, jnp.int32) + d
                    vals = plsc.load_gather(rows_v.at[b], [tvec, dvec])
                    trans_v[p, d, pl.ds(tc * LANES, LANES)] = vals
                return 0

            lax.fori_loop(0, dim, drow, 0)

        for b in range(G):
            fire_gather(b, b)

        def group(g, _):
            for b in range(K):
                j = g * K + b
                wait_gather(j, b)

                @pl.when(j + G < n)
                def _():
                    fire_gather(j + G, (b + G) % K)

                @pl.when(j - 2 >= 0)
                def _():
                    wait_writes(j - 2, b % 2)

                transpose_block(b, b % 2)
                fire_writes(j, b % 2)

            return 0

        lax.fori_loop(0, n // K, group, 0)
        wait_writes(n - 2, 0)
        wait_writes(n - 1, 1)

    return emb_kernel(idx2d, table)


def kernel(token_ids, embedding_matrix):
    n_tokens, seq = token_ids.shape
    dim = embedding_matrix.shape[1]
    tblocks = n_tokens // TB
    idx2d = token_ids.T.astype(jnp.int32).reshape(seq * tblocks, TB)
    out5 = _embedding_lookup(idx2d, embedding_matrix, seq, tblocks)
    return out5.transpose(2, 4, 0, 1, 3).reshape(n_tokens, seq, dim)


# speedup vs baseline: 2.4254x; 2.4254x over previous
import functools

import jax
import jax.numpy as jnp
from jax import lax
from jax.experimental import pallas as pl
from jax.experimental.pallas import tpu as pltpu
from jax.experimental.pallas import tpu_sc as plsc

NUM_CORES = 2
NUM_SUBCORES = 16
NUM_WORKERS = NUM_CORES * NUM_SUBCORES
TB = 128
K = 8
G = 4
LANES = 16


@functools.partial(jax.jit, static_argnums=(2, 3))
def _embedding_lookup(idx2d, table, seq, tblocks):
    dim = table.shape[1]
    dg = dim // 8
    n = idx2d.shape[0] // NUM_WORKERS
    assert n % K == 0
    mesh = plsc.VectorSubcoreMesh(core_axis_name="c", subcore_axis_name="s")

    @functools.partial(
        pl.kernel,
        mesh=mesh,
        out_type=jax.ShapeDtypeStruct((seq, dg, tblocks, 8, TB), jnp.float32),
        scratch_types=[
            pltpu.VMEM((n, TB), jnp.int32),
            pltpu.VMEM((K, TB, dim), jnp.float32),
            pltpu.VMEM((2, dim, TB), jnp.float32),
            pltpu.SemaphoreType.DMA((K,)),
            pltpu.SemaphoreType.DMA((2,)),
        ],
        compiler_params=pltpu.CompilerParams(
            use_tc_tiling_on_sc=False, needs_layout_passes=False
        ),
    )
    def emb_kernel(idx_hbm, table_hbm, out_hbm, idx_v, rows_v, trans_v,
                   gsem, wsem):
        wid = lax.axis_index("s") * NUM_CORES + lax.axis_index("c")
        blk_base = wid * n
        pltpu.sync_copy(idx_hbm.at[pl.ds(blk_base, n)], idx_v)

        def fire_gather(j, b):
            pltpu.async_copy(table_hbm.at[idx_v.at[j]], rows_v.at[b], gsem.at[b])

        def wait_gather(j, b):
            pltpu.make_async_copy(
                table_hbm.at[idx_v.at[j]], rows_v.at[b], gsem.at[b]
            ).wait()

        def write_parts(j, p):
            blk = blk_base + j
            s = blk // tblocks
            tb = blk % tblocks
            return [
                (trans_v.at[p, pl.ds(8 * g, 8)], out_hbm.at[s, g, tb])
                for g in range(dg)
            ]

        def fire_writes(j, p):
            for src, dst in write_parts(j, p):
                pltpu.async_copy(src, dst, wsem.at[p])

        def wait_writes(j, p):
            for src, dst in write_parts(j, p):
                pltpu.make_async_copy(src, dst, wsem.at[p]).wait()

        lane_iota = lax.iota(jnp.int32, LANES)

        def transpose_block(b, p):
            @functools.partial(plsc.parallel_loop, 0, dim, unroll=4)
            def _(d):
                for tc in range(TB // LANES):
                    tvec = lane_iota + (tc * LANES)
                    dvec = jnp.full((LANES,), 0, jnp.int32) + d
                    vals = plsc.load_gather(rows_v.at[b], [tvec, dvec])
                    trans_v[p, d, pl.ds(tc * LANES, LANES)] = vals

        for b in range(G):
            fire_gather(b, b)

        def group(g, _):
            for b in range(K):
                j = g * K + b
                wait_gather(j, b)

                @pl.when(j + G < n)
                def _():
                    fire_gather(j + G, (b + G) % K)

                @pl.when(j - 2 >= 0)
                def _():
                    wait_writes(j - 2, b % 2)

                transpose_block(b, b % 2)
                fire_writes(j, b % 2)

            return 0

        lax.fori_loop(0, n // K, group, 0)
        wait_writes(n - 2, 0)
        wait_writes(n - 1, 1)

    return emb_kernel(idx2d, table)


def kernel(token_ids, embedding_matrix):
    n_tokens, seq = token_ids.shape
    dim = embedding_matrix.shape[1]
    tblocks = n_tokens // TB
    idx2d = token_ids.T.astype(jnp.int32).reshape(seq * tblocks, TB)
    out5 = _embedding_lookup(idx2d, embedding_matrix, seq, tblocks)
    return out5.transpose(2, 4, 0, 1, 3).reshape(n_tokens, seq, dim)
